# Initial kernel scaffold; baseline (speedup 1.0000x reference)
import jax
import jax.numpy as jnp
from jax.experimental import pallas as pl


def kernel(x, W0, b0, W1, b1, topk):
    raise NotImplementedError("write your pallas kernel here")


# speedup vs baseline: 43.5261x; 43.5261x over previous
import functools

import jax
import jax.numpy as jnp
from jax import lax
from jax.experimental import pallas as pl
from jax.experimental.pallas import tpu as pltpu


def _stage_a_body(num_groups, k, x_ref, w0_ref, b0_ref, sel_ref, fidx_ref,
                  psel_ref):
    batch = x_ref.shape[0]
    x = x_ref[...]
    w0 = w0_ref[...]
    logits0 = lax.dot_general(x, w0, (((1,), (1,)), ((), ())),
                              preferred_element_type=jnp.float32)
    logits0 = logits0 + b0_ref[...]
    iota = lax.broadcasted_iota(jnp.int32, (batch, num_groups), 1)
    neg = jnp.float32(-3e38)
    bigi = jnp.int32(2**30)

    def pick(_, carry):
        vals, selmask = carry
        m = jnp.max(vals, axis=1, keepdims=True)
        first = jnp.min(jnp.where(vals == m, iota, bigi), axis=1,
                        keepdims=True)
        hit = iota == first
        return jnp.where(hit, neg, vals), jnp.where(hit, 1, selmask)

    _, selmask = lax.fori_loop(
        0, k, pick,
        (logits0, jnp.zeros((batch, num_groups), jnp.int32)))

    lane_k = lax.broadcasted_iota(jnp.int32, (batch, k), 1)

    def extract(j, carry):
        cur, selv, lsel = carry
        m = jnp.min(cur, axis=1, keepdims=True)
        hit = cur == m
        l = jnp.max(jnp.where(hit, logits0, neg), axis=1, keepdims=True)
        selv = jnp.where(lane_k == j, m, selv)
        lsel = jnp.where(lane_k == j, l, lsel)
        return jnp.where(hit, bigi, cur), selv, lsel

    cur0 = jnp.where(selmask == 1, iota, bigi)
    _, selv, lsel = lax.fori_loop(
        0, k, extract,
        (cur0, jnp.zeros((batch, k), jnp.int32),
         jnp.zeros((batch, k), jnp.float32)))
    sel_ref[...] = selv
    row = lax.broadcasted_iota(jnp.int32, (batch, k), 0)
    fidx_ref[...] = row * num_groups + selv
    psel_ref[...] = 1.0 / (1.0 + jnp.exp(-lsel))


def _stage_b_body(gt, branch, x_ref, w1_ref, b1_ref, out_ref):
    x = x_ref[...]
    for j in range(gt):
        w = w1_ref[pl.ds(j * branch, branch), :]
        r = lax.dot_general(x, w, (((1,), (1,)), ((), ())),
                            preferred_element_type=jnp.float32)
        r = r + b1_ref[0, :, pl.ds(j * branch, branch)]
        out_ref[:, j, 0:branch] = r


def _gather_body(k, sel_ref, table_ref, out_ref):
    b = pl.program_id(0)

    def body(j, carry):
        g = sel_ref[b, j]
        out_ref[0, pl.ds(j, 1), :] = table_ref[0, pl.ds(g, 1), :]
        return carry

    lax.fori_loop(0, k, body, 0)


def _combine_body(branch, g_ref, p_ref, s_ref, probs_ref, child_ref):
    g = g_ref[...]
    sig = 1.0 / (1.0 + jnp.exp(-g))
    probs_ref[...] = p_ref[...] * sig
    child_ref[...] = s_ref[...] * branch + lax.broadcasted_iota(
        jnp.int32, g_ref.shape, 1)


def kernel(x, W0, b0, W1, b1, topk):
    batch, d = x.shape
    num_groups = W0.shape[0]
    num_leaves = W1.shape[0]
    branch = num_leaves // num_groups
    k = min(64, num_groups)
    gpad = 128
    del topk

    sel, fidx, psel = pl.pallas_call(
        functools.partial(_stage_a_body, num_groups, k),
        grid=(1,),
        in_specs=[
            pl.BlockSpec((batch, d), lambda i: (0, 0)),
            pl.BlockSpec((num_groups, d), lambda i: (0, 0)),
            pl.BlockSpec((1, num_groups), lambda i: (0, 0)),
        ],
        out_specs=[
            pl.BlockSpec((batch, k), lambda i: (0, 0)),
            pl.BlockSpec((batch, k), lambda i: (0, 0)),
            pl.BlockSpec((batch, k), lambda i: (0, 0)),
        ],
        out_shape=[
            jax.ShapeDtypeStruct((batch, k), jnp.int32),
            jax.ShapeDtypeStruct((batch, k), jnp.int32),
            jax.ShapeDtypeStruct((batch, k), jnp.float32),
        ],
    )(x, W0, b0.reshape(1, num_groups))

    gt = 40
    table = pl.pallas_call(
        functools.partial(_stage_b_body, gt, branch),
        grid=(num_groups // gt,),
        in_specs=[
            pl.BlockSpec((batch, d), lambda i: (0, 0)),
            pl.BlockSpec((gt * branch, d), lambda i: (i, 0)),
            pl.BlockSpec((1, 1, gt * branch), lambda i: (i, 0, 0)),
        ],
        out_specs=pl.BlockSpec((batch, gt, gpad), lambda i: (0, i, 0)),
        out_shape=jax.ShapeDtypeStruct((batch, num_groups, gpad),
                                       jnp.float32),
    )(x, W1, b1.reshape(num_groups // gt, 1, gt * branch))

    gathered = pl.pallas_call(
        functools.partial(_gather_body, k),
        grid=(batch,),
        in_specs=[
            pl.BlockSpec(memory_space=pltpu.SMEM),
            pl.BlockSpec((1, num_groups, gpad), lambda b: (b, 0, 0)),
        ],
        out_specs=pl.BlockSpec((1, k, gpad), lambda b: (b, 0, 0)),
        out_shape=jax.ShapeDtypeStruct((batch, k, gpad), jnp.float32),
    )(sel, table)

    nrows = batch * k
    probs_pad, child_pad = pl.pallas_call(
        functools.partial(_combine_body, branch),
        grid=(1,),
        in_specs=[
            pl.BlockSpec((nrows, gpad), lambda i: (0, 0)),
            pl.BlockSpec((nrows, 1), lambda i: (0, 0)),
            pl.BlockSpec((nrows, 1), lambda i: (0, 0)),
        ],
        out_specs=[
            pl.BlockSpec((nrows, gpad), lambda i: (0, 0)),
            pl.BlockSpec((nrows, gpad), lambda i: (0, 0)),
        ],
        out_shape=[
            jax.ShapeDtypeStruct((nrows, gpad), jnp.float32),
            jax.ShapeDtypeStruct((nrows, gpad), jnp.int32),
        ],
    )(gathered.reshape(nrows, gpad), psel.reshape(nrows, 1),
      sel.reshape(nrows, 1))

    probs = probs_pad[:, :branch].reshape(batch, k * branch)
    children = child_pad[:, :branch].reshape(batch, k * branch)
    mask = jnp.ones((batch, k * branch), dtype=bool)
    return probs, children, mask


# speedup vs baseline: 45.4996x; 1.0453x over previous
import functools

import jax
import jax.numpy as jnp
from jax import lax
from jax.experimental import pallas as pl
from jax.experimental.pallas import tpu as pltpu
from jax.experimental.pallas import tpu_sc as plsc


def _stage_a_body(num_groups, k, x_ref, w0_ref, b0_ref, sel_ref, fidx_ref,
                  psel_ref):
    batch = x_ref.shape[0]
    x = x_ref[...]
    w0 = w0_ref[...]
    logits0 = lax.dot_general(x, w0, (((1,), (1,)), ((), ())),
                              preferred_element_type=jnp.float32)
    logits0 = logits0 + b0_ref[...]
    iota = lax.broadcasted_iota(jnp.int32, (batch, num_groups), 1)
    neg = jnp.float32(-3e38)
    bigi = jnp.int32(2**30)

    def pick(_, carry):
        vals, selmask = carry
        m = jnp.max(vals, axis=1, keepdims=True)
        first = jnp.min(jnp.where(vals == m, iota, bigi), axis=1,
                        keepdims=True)
        hit = iota == first
        return jnp.where(hit, neg, vals), jnp.where(hit, 1, selmask)

    _, selmask = lax.fori_loop(
        0, k, pick,
        (logits0, jnp.zeros((batch, num_groups), jnp.int32)))

    lane_k = lax.broadcasted_iota(jnp.int32, (batch, k), 1)

    def extract(j, carry):
        cur, selv, lsel = carry
        m = jnp.min(cur, axis=1, keepdims=True)
        hit = cur == m
        l = jnp.max(jnp.where(hit, logits0, neg), axis=1, keepdims=True)
        selv = jnp.where(lane_k == j, m, selv)
        lsel = jnp.where(lane_k == j, l, lsel)
        return jnp.where(hit, bigi, cur), selv, lsel

    cur0 = jnp.where(selmask == 1, iota, bigi)
    _, selv, lsel = lax.fori_loop(
        0, k, extract,
        (cur0, jnp.zeros((batch, k), jnp.int32),
         jnp.zeros((batch, k), jnp.float32)))
    sel_ref[...] = selv
    row = lax.broadcasted_iota(jnp.int32, (batch, k), 0)
    fidx_ref[...] = row * num_groups + selv
    psel_ref[...] = 1.0 / (1.0 + jnp.exp(-lsel))


def _stage_b_body(gt, branch, x_ref, w1_ref, b1_ref, out_ref):
    x = x_ref[...]
    for j in range(gt):
        w = w1_ref[pl.ds(j * branch, branch), :]
        r = lax.dot_general(x, w, (((1,), (1,)), ((), ())),
                            preferred_element_type=jnp.float32)
        r = r + b1_ref[0, :, pl.ds(j * branch, branch)]
        out_ref[:, j, 0:branch] = r


def _sc_gather(fidx, table2d):
    nrows = fidx.shape[0]
    info = plsc.get_sparse_core_info()
    nc, ns = info.num_cores, info.num_subcores
    nw = nc * ns
    bpw = nrows // nw
    mesh = plsc.VectorSubcoreMesh(core_axis_name="c", subcore_axis_name="s")

    @functools.partial(
        pl.kernel,
        out_type=jax.ShapeDtypeStruct((nrows, table2d.shape[1]),
                                      jnp.float32),
        mesh=mesh,
        scratch_types=[
            pltpu.VMEM((bpw,), jnp.int32),
            pltpu.VMEM((bpw, table2d.shape[1]), jnp.float32),
            pltpu.SemaphoreType.DMA,
        ],
    )
    def k(fidx_hbm, table_hbm, out_hbm, idx_v, rows_v, sem):
        wid = lax.axis_index("s") * nc + lax.axis_index("c")
        base = wid * bpw
        pltpu.sync_copy(fidx_hbm.at[pl.ds(base, bpw)], idx_v)
        pltpu.async_copy(table_hbm.at[idx_v], rows_v, sem).wait()
        pltpu.sync_copy(rows_v, out_hbm.at[pl.ds(base, bpw)])

    return k(fidx, table2d)


def _combine_body(branch, g_ref, p_ref, s_ref, probs_ref, child_ref):
    g = g_ref[...]
    sig = 1.0 / (1.0 + jnp.exp(-g))
    probs_ref[...] = p_ref[...] * sig
    child_ref[...] = s_ref[...] * branch + lax.broadcasted_iota(
        jnp.int32, g_ref.shape, 1)


def kernel(x, W0, b0, W1, b1, topk):
    batch, d = x.shape
    num_groups = W0.shape[0]
    num_leaves = W1.shape[0]
    branch = num_leaves // num_groups
    k = min(64, num_groups)
    gpad = 128
    del topk

    sel, fidx, psel = pl.pallas_call(
        functools.partial(_stage_a_body, num_groups, k),
        grid=(1,),
        in_specs=[
            pl.BlockSpec((batch, d), lambda i: (0, 0)),
            pl.BlockSpec((num_groups, d), lambda i: (0, 0)),
            pl.BlockSpec((1, num_groups), lambda i: (0, 0)),
        ],
        out_specs=[
            pl.BlockSpec((batch, k), lambda i: (0, 0)),
            pl.BlockSpec((batch, k), lambda i: (0, 0)),
            pl.BlockSpec((batch, k), lambda i: (0, 0)),
        ],
        out_shape=[
            jax.ShapeDtypeStruct((batch, k), jnp.int32),
            jax.ShapeDtypeStruct((batch, k), jnp.int32),
            jax.ShapeDtypeStruct((batch, k), jnp.float32),
        ],
    )(x, W0, b0.reshape(1, num_groups))

    gt = 40
    table = pl.pallas_call(
        functools.partial(_stage_b_body, gt, branch),
        grid=(num_groups // gt,),
        in_specs=[
            pl.BlockSpec((batch, d), lambda i: (0, 0)),
            pl.BlockSpec((gt * branch, d), lambda i: (i, 0)),
            pl.BlockSpec((1, 1, gt * branch), lambda i: (i, 0, 0)),
        ],
        out_specs=pl.BlockSpec((batch, gt, gpad), lambda i: (0, i, 0)),
        out_shape=jax.ShapeDtypeStruct((batch, num_groups, gpad),
                                       jnp.float32),
    )(x, W1, b1.reshape(num_groups // gt, 1, gt * branch))

    nrows = batch * k
    gathered = _sc_gather(fidx.reshape(nrows),
                          table.reshape(batch * num_groups, gpad))

    probs_pad, child_pad = pl.pallas_call(
        functools.partial(_combine_body, branch),
        grid=(1,),
        in_specs=[
            pl.BlockSpec((nrows, gpad), lambda i: (0, 0)),
            pl.BlockSpec((nrows, 1), lambda i: (0, 0)),
            pl.BlockSpec((nrows, 1), lambda i: (0, 0)),
        ],
        out_specs=[
            pl.BlockSpec((nrows, gpad), lambda i: (0, 0)),
            pl.BlockSpec((nrows, gpad), lambda i: (0, 0)),
        ],
        out_shape=[
            jax.ShapeDtypeStruct((nrows, gpad), jnp.float32),
            jax.ShapeDtypeStruct((nrows, gpad), jnp.int32),
        ],
    )(gathered, psel.reshape(nrows, 1), sel.reshape(nrows, 1))

    probs = probs_pad[:, :branch].reshape(batch, k * branch)
    children = child_pad[:, :branch].reshape(batch, k * branch)
    mask = jnp.ones((batch, k * branch), dtype=bool)
    return probs, children, mask
